# baseline (speedup 1.0000x reference)
import functools
import math

import jax
import jax.numpy as jnp
from jax import lax
from jax.experimental import pallas as pl
from jax.experimental.pallas import tpu as pltpu
from jax.experimental.pallas import tpu_sc as plsc

LANES = 16
NUM_CORES = 2
NUM_SUBCORES = 16
NUM_WORKERS = NUM_CORES * NUM_SUBCORES


def _make_kernel(S, B, D):
    s_per_w = S // NUM_WORKERS
    chunk = 32
    n_chunk = (s_per_w * B) // chunk
    s_per_chunk = chunk // B
    s_half = s_per_chunk // 2
    jblk = D // 128
    mv = 128 // LANES
    kv = D // LANES
    scale = math.sqrt(D)

    mesh = plsc.VectorSubcoreMesh(core_axis_name="c", subcore_axis_name="s")

    @functools.partial(
        pl.kernel,
        mesh=mesh,
        out_type=jax.ShapeDtypeStruct((S, jblk, B, 128), jnp.float32),
        scratch_types=[
            pltpu.VMEM((n_chunk, chunk), jnp.int32),
            pltpu.VMEM((s_per_w, 1, D), jnp.float32),
            pltpu.VMEM((chunk, D), jnp.float32),
            pltpu.VMEM((chunk, D), jnp.float32),
            pltpu.VMEM((s_half, jblk, B, 128), jnp.float32),
            pltpu.VMEM((s_half, jblk, B, 128), jnp.float32),
            pltpu.SemaphoreType.DMA,
            pltpu.SemaphoreType.DMA,
            pltpu.SemaphoreType.DMA,
            pltpu.SemaphoreType.DMA,
        ],
    )
    def k(x_hbm, pe_hbm, table_hbm, out_hbm,
          idx_v, pe_v, g0, g1, o0, o1, gs0, gs1, os0, os1):
        wid = lax.axis_index("s") * NUM_CORES + lax.axis_index("c")
        sbase = wid * s_per_w

        pltpu.sync_copy(x_hbm.at[wid], idx_v)
        pltpu.sync_copy(pe_hbm.at[pl.ds(sbase, s_per_w)], pe_v)

        gbufs, obufs = (g0, g1), (o0, o1)
        gsems, osems = (gs0, gs1), (os0, os1)
        pltpu.async_copy(table_hbm.at[idx_v.at[0]], g0, gs0)
        pltpu.async_copy(table_hbm.at[idx_v.at[1]], g1, gs1)

        def pair_body(p, _):
            for par in range(2):
                c = 2 * p + par
                gb = gbufs[par]
                pltpu.make_async_copy(
                    table_hbm.at[idx_v.at[c]], gb, gsems[par]
                ).wait()

                for h in range(2):
                    ob = obufs[h]
                    s_off = c * s_per_chunk + h * s_half
                    r_off = h * s_half * B

                    @pl.when(c > 0)
                    def _drain(ob=ob, h=h):
                        pltpu.make_async_copy(
                            ob, out_hbm.at[pl.ds(0, s_half)], osems[h]
                        ).wait()

                    @plsc.parallel_loop(0, s_half)
                    def sloop(sl, gb=gb, ob=ob, s_off=s_off, r_off=r_off):
                        @plsc.parallel_loop(0, kv, unroll=12)
                        def jloop(jm, sl=sl, gb=gb, ob=ob,
                                  s_off=s_off, r_off=r_off):
                            j = jm // mv
                            m = jm % mv
                            col = jm * LANES
                            pv = pe_v[s_off + sl, 0, pl.ds(col, LANES)]
                            for b in range(B):
                                ob[sl, j, b, pl.ds(m * LANES, LANES)] = (
                                    gb[r_off + sl * B + b, pl.ds(col, LANES)]
                                    * scale + pv
                                )

                    pltpu.async_copy(
                        ob, out_hbm.at[pl.ds(sbase + s_off, s_half)], osems[h]
                    )

                @pl.when(c + 2 < n_chunk)
                def _prefetch(gb=gb, par=par, c=c):
                    pltpu.async_copy(
                        table_hbm.at[idx_v.at[c + 2]], gb, gsems[par]
                    )

            return 0

        lax.fori_loop(0, n_chunk // 2, pair_body, 0)

        pltpu.make_async_copy(o0, out_hbm.at[pl.ds(0, s_half)], os0).wait()
        pltpu.make_async_copy(o1, out_hbm.at[pl.ds(0, s_half)], os1).wait()

    return k


@jax.jit
def kernel(x, table, pe):
    S, B = x.shape
    V, D = table.shape
    x_w = x.astype(jnp.int32).reshape(NUM_WORKERS, -1, 32)
    k = _make_kernel(S, B, D)
    out4 = k(x_w, pe, table)
    return jnp.swapaxes(out4, 1, 2).reshape(S, B, D)


# speedup vs baseline: 1.3540x; 1.3540x over previous
import functools
import math

import jax
import jax.numpy as jnp
from jax import lax
from jax.experimental import pallas as pl
from jax.experimental.pallas import tpu as pltpu
from jax.experimental.pallas import tpu_sc as plsc

LANES = 16
NUM_CORES = 2
NUM_SUBCORES = 16
NUM_WORKERS = NUM_CORES * NUM_SUBCORES


def _make_kernel(S, B, D):
    s_per_w = S // NUM_WORKERS
    chunk = 32
    n_chunk = (s_per_w * B) // chunk
    s_per_chunk = chunk // B
    s_half = s_per_chunk // 2
    jblk = D // 128
    mv = 128 // LANES
    kv = D // LANES
    scale = math.sqrt(D)

    mesh = plsc.VectorSubcoreMesh(core_axis_name="c", subcore_axis_name="s")

    @functools.partial(
        pl.kernel,
        mesh=mesh,
        out_type=jax.ShapeDtypeStruct((S, jblk, B, 128), jnp.float32),
        scratch_types=[
            pltpu.VMEM((n_chunk, chunk), jnp.int32),
            pltpu.VMEM((s_per_w, 1, D), jnp.float32),
            pltpu.VMEM((chunk, D), jnp.float32),
            pltpu.VMEM((chunk, D), jnp.float32),
            pltpu.VMEM((s_half, jblk, B, 128), jnp.float32),
            pltpu.VMEM((s_half, jblk, B, 128), jnp.float32),
            pltpu.SemaphoreType.DMA,
            pltpu.SemaphoreType.DMA,
            pltpu.SemaphoreType.DMA,
            pltpu.SemaphoreType.DMA,
        ],
    )
    def k(x_hbm, pe_hbm, table_hbm, out_hbm,
          idx_v, pe_v, g0, g1, o0, o1, gs0, gs1, os0, os1):
        wid = lax.axis_index("s") * NUM_CORES + lax.axis_index("c")
        sbase = wid * s_per_w

        pltpu.sync_copy(x_hbm.at[wid], idx_v)
        pltpu.sync_copy(pe_hbm.at[pl.ds(sbase, s_per_w)], pe_v)

        gbufs, obufs = (g0, g1), (o0, o1)
        gsems, osems = (gs0, gs1), (os0, os1)
        pltpu.async_copy(table_hbm.at[idx_v.at[0]], g0, gs0)
        pltpu.async_copy(table_hbm.at[idx_v.at[1]], g1, gs1)

        def pair_body(p, _):
            for par in range(2):
                c = 2 * p + par
                gb = gbufs[par]
                pltpu.make_async_copy(
                    table_hbm.at[idx_v.at[c]], gb, gsems[par]
                ).wait()

                for h in range(2):
                    ob = obufs[h]
                    s_off = c * s_per_chunk + h * s_half
                    r_off = h * s_half * B

                    @pl.when(c > 0)
                    def _drain(ob=ob, h=h):
                        pltpu.make_async_copy(
                            ob, out_hbm.at[pl.ds(0, s_half)], osems[h]
                        ).wait()

                    @plsc.parallel_loop(0, s_half)
                    def sloop(sl, gb=gb, ob=ob, s_off=s_off, r_off=r_off):
                        @plsc.parallel_loop(0, kv, unroll=8)
                        def jloop(jm, sl=sl, gb=gb, ob=ob,
                                  s_off=s_off, r_off=r_off):
                            j = jm // mv
                            m = jm % mv
                            col = jm * LANES
                            pv = pe_v[s_off + sl, 0, pl.ds(col, LANES)]
                            for b in range(B):
                                ob[sl, j, b, pl.ds(m * LANES, LANES)] = (
                                    gb[r_off + sl * B + b, pl.ds(col, LANES)]
                                    * scale + pv
                                )

                    pltpu.async_copy(
                        ob, out_hbm.at[pl.ds(sbase + s_off, s_half)], osems[h]
                    )

                @pl.when(c + 2 < n_chunk)
                def _prefetch(gb=gb, par=par, c=c):
                    pltpu.async_copy(
                        table_hbm.at[idx_v.at[c + 2]], gb, gsems[par]
                    )

            return 0

        lax.fori_loop(0, n_chunk // 2, pair_body, 0)

        pltpu.make_async_copy(o0, out_hbm.at[pl.ds(0, s_half)], os0).wait()
        pltpu.make_async_copy(o1, out_hbm.at[pl.ds(0, s_half)], os1).wait()

    return k


@jax.jit
def kernel(x, table, pe):
    S, B = x.shape
    V, D = table.shape
    x_w = x.astype(jnp.int32).reshape(NUM_WORKERS, -1, 32)
    k = _make_kernel(S, B, D)
    out4 = k(x_w, pe, table)
    return jnp.swapaxes(out4, 1, 2).reshape(S, B, D)
